# baseline (speedup 1.0000x reference)
import functools

import jax
import jax.numpy as jnp
from jax import lax
from jax.experimental import pallas as pl
from jax.experimental.pallas import tpu as pltpu
from jax.experimental.pallas import tpu_sc as plsc

_A, _B, _C, _D = 16, 128, 128, 256
_EPS = 1e-6
_ROWS = _A * _B

_SC_ROWS = 512
_TC_ROWS = _ROWS - _SC_ROWS
_NW = 32
_RPW = _SC_ROWS // _NW

_CH = 32
_NCH = _TC_ROWS // _CH
_K = 4


def _recip_kernel(pe0_ref, pe1_ref, pe2_ref, recip_ref, pe01_ref):
    pe0 = pe0_ref[:, :]
    pe1 = pe1_ref[:, :]
    pe2 = pe2_ref[:, :]
    pe01 = (pe0[:, None, :] + pe1[None, :, :]).reshape(_ROWS, _D)
    pe01_ref[:, :] = pe01
    n01 = jnp.sum(pe01 * pe01, axis=-1, keepdims=True)
    n2 = jnp.sum(pe2 * pe2, axis=-1)
    dots = jax.lax.dot_general(
        pe01.astype(jnp.bfloat16),
        pe2.astype(jnp.bfloat16),
        (((1,), (1,)), ((), ())),
        preferred_element_type=jnp.float32,
    )
    ssq = n01 + 2.0 * dots + n2[None, :]
    recip_ref[:, :] = 1.0 / (_EPS + jnp.sqrt(ssq) * (1.0 / 16.0))


def _wide_kernel(pe0_ref, pe1_ref, pe2_ref, recip_ref, out_ref,
                 pe01_ref, buf_ref, sem_ref):
    pe01_ref[:, :] = (
        pe0_ref[:, :][:, None, :] + pe1_ref[:, :][None, :, :]
    ).reshape(_ROWS, _D)
    pe2 = pe2_ref[:, :]

    def body(i, carry):
        slot = jax.lax.rem(i, _K)

        @pl.when(i >= _K)
        def _():
            pltpu.make_async_copy(
                buf_ref.at[slot],
                out_ref.at[pl.ds((i - _K) * _CH, _CH)],
                sem_ref.at[slot],
            ).wait()

        pe01_blk = pe01_ref[pl.ds(i * _CH, _CH), :]
        r = recip_ref[pl.ds(i * _CH, _CH), :]
        buf_ref[slot] = (pe01_blk[:, None, :] + pe2[None, :, :]) * r[:, :, None]
        pltpu.make_async_copy(
            buf_ref.at[slot],
            out_ref.at[pl.ds(i * _CH, _CH)],
            sem_ref.at[slot],
        ).start()
        return carry

    jax.lax.fori_loop(0, _NCH, body, 0)

    def drain(j, carry):
        slot = jax.lax.rem(j, _K)
        pltpu.make_async_copy(
            buf_ref.at[slot],
            out_ref.at[pl.ds(j * _CH, _CH)],
            sem_ref.at[slot],
        ).wait()
        return carry

    jax.lax.fori_loop(_NCH - _K, _NCH, drain, 0)


def _sc_body(pe01_hbm, pe2_hbm, recip_hbm, out_hbm,
             pe2_v, pe01_v, recip_v, buf_v, sems):
    wid = lax.axis_index("s") * 2 + lax.axis_index("c")
    base = wid * _RPW

    pltpu.sync_copy(pe2_hbm, pe2_v)
    pltpu.sync_copy(pe01_hbm.at[pl.ds(base, _RPW)], pe01_v)
    pltpu.sync_copy(recip_hbm.at[pl.ds(base * _C, _RPW * _C)], recip_v)

    def row_body(i, carry):
        slot = lax.rem(i, 2)

        @pl.when(i >= 2)
        def _():
            pltpu.make_async_copy(
                buf_v.at[slot], out_hbm.at[base + i - 2], sems.at[slot]
            ).wait()

        def c16_body(cb, carry2):
            c0 = cb * 16
            rvec = recip_v[pl.ds(i * _C + c0, 16)]
            for l in range(16):
                rb = jnp.full((16,), rvec[l])
                for j in range(_D // 16):
                    sl = pl.ds(j * 16, 16)
                    buf_v[slot, c0 + l, sl] = (pe01_v[i, sl] + pe2_v[c0 + l, sl]) * rb
            return carry2

        lax.fori_loop(0, _C // 16, c16_body, 0)
        pltpu.make_async_copy(
            buf_v.at[slot], out_hbm.at[base + i], sems.at[slot]
        ).start()
        return carry

    lax.fori_loop(0, _RPW, row_body, 0)

    def drain(j, carry):
        pltpu.make_async_copy(
            buf_v.at[lax.rem(j, 2)], out_hbm.at[base + j], sems.at[lax.rem(j, 2)]
        ).wait()
        return carry

    lax.fori_loop(_RPW - 2, _RPW, drain, 0)


_sc_kernel = functools.partial(
    pl.kernel,
    out_type=jax.ShapeDtypeStruct((_SC_ROWS, _C, _D), jnp.float32),
    mesh=plsc.VectorSubcoreMesh(core_axis_name="c", subcore_axis_name="s"),
    scratch_types=[
        pltpu.MemorySpace.VMEM((_C, _D), jnp.float32),
        pltpu.MemorySpace.VMEM((_RPW, _D), jnp.float32),
        pltpu.MemorySpace.VMEM((_RPW * _C,), jnp.float32),
        pltpu.MemorySpace.VMEM((2, _C, _D), jnp.float32),
        pltpu.SemaphoreType.DMA((2,)),
    ],
)(_sc_body)


def kernel(pos_embed_0, pos_embed_1, pos_embed_2, axial0, axial1, axial2):
    pe0 = pos_embed_0[:_A]
    pe1 = pos_embed_1[:_B]
    pe2 = pos_embed_2[:_C]

    recip, pe01 = pl.pallas_call(
        _recip_kernel,
        out_shape=[
            jax.ShapeDtypeStruct((_ROWS, _C), jnp.float32),
            jax.ShapeDtypeStruct((_ROWS, _D), jnp.float32),
        ],
    )(pe0, pe1, pe2)

    out_sc = _sc_kernel(
        pe01[_TC_ROWS:], pe2, recip[_TC_ROWS:].reshape(_SC_ROWS * _C)
    )

    out_tc = pl.pallas_call(
        _wide_kernel,
        in_specs=[
            pl.BlockSpec(memory_space=pltpu.MemorySpace.VMEM),
            pl.BlockSpec(memory_space=pltpu.MemorySpace.VMEM),
            pl.BlockSpec(memory_space=pltpu.MemorySpace.VMEM),
            pl.BlockSpec(memory_space=pltpu.MemorySpace.VMEM),
        ],
        out_specs=pl.BlockSpec(memory_space=pltpu.MemorySpace.HBM),
        out_shape=jax.ShapeDtypeStruct((_TC_ROWS, _C, _D), jnp.float32),
        scratch_shapes=[
            pltpu.MemorySpace.VMEM((_ROWS, _D), jnp.float32),
            pltpu.MemorySpace.VMEM((_K, _CH, _C, _D), jnp.float32),
            pltpu.SemaphoreType.DMA((_K,)),
        ],
    )(pe0, pe1, pe2, recip)

    out = jnp.concatenate([out_tc, out_sc], axis=0)
    return out.reshape(_A, _B, _C, _D)


# speedup vs baseline: 1.6103x; 1.6103x over previous
import functools

import jax
import jax.numpy as jnp
from jax import lax
from jax.experimental import pallas as pl
from jax.experimental.pallas import tpu as pltpu
from jax.experimental.pallas import tpu_sc as plsc

_A, _B, _C, _D = 16, 128, 128, 256
_EPS = 1e-6
_ROWS = _A * _B

_SC_ROWS = 512
_TC_ROWS = _ROWS - _SC_ROWS
_NW = 32
_RPW = _SC_ROWS // _NW

_CH = 32
_NCH = _TC_ROWS // _CH
_K = 4


def _recip_kernel(pe0_ref, pe1_ref, pe2_ref, recip_ref, pe01_ref):
    pe0 = pe0_ref[:, :]
    pe1 = pe1_ref[:, :]
    pe2 = pe2_ref[:, :]
    pe01 = (pe0[:, None, :] + pe1[None, :, :]).reshape(_ROWS, _D)
    pe01_ref[:, :] = pe01
    n01 = jnp.sum(pe01 * pe01, axis=-1, keepdims=True)
    n2 = jnp.sum(pe2 * pe2, axis=-1)
    dots = jax.lax.dot_general(
        pe01.astype(jnp.bfloat16),
        pe2.astype(jnp.bfloat16),
        (((1,), (1,)), ((), ())),
        preferred_element_type=jnp.float32,
    )
    ssq = n01 + 2.0 * dots + n2[None, :]
    recip_ref[:, :] = 1.0 / (_EPS + jnp.sqrt(ssq) * (1.0 / 16.0))


def _wide_kernel(pe0_ref, pe1_ref, pe2_ref, recip_ref, out_ref,
                 pe01_ref, buf_ref, sem_ref):
    pe01_ref[:, :] = (
        pe0_ref[:, :][:, None, :] + pe1_ref[:, :][None, :, :]
    ).reshape(_ROWS, _D)
    pe2 = pe2_ref[:, :]

    def body(i, carry):
        slot = jax.lax.rem(i, _K)

        @pl.when(i >= _K)
        def _():
            pltpu.make_async_copy(
                buf_ref.at[slot],
                out_ref.at[pl.ds((i - _K) * _CH, _CH)],
                sem_ref.at[slot],
            ).wait()

        pe01_blk = pe01_ref[pl.ds(i * _CH, _CH), :]
        r = recip_ref[pl.ds(i * _CH, _CH), :]
        buf_ref[slot] = (pe01_blk[:, None, :] + pe2[None, :, :]) * r[:, :, None]
        pltpu.make_async_copy(
            buf_ref.at[slot],
            out_ref.at[pl.ds(i * _CH, _CH)],
            sem_ref.at[slot],
        ).start()
        return carry

    jax.lax.fori_loop(0, _NCH, body, 0)

    def drain(j, carry):
        slot = jax.lax.rem(j, _K)
        pltpu.make_async_copy(
            buf_ref.at[slot],
            out_ref.at[pl.ds(j * _CH, _CH)],
            sem_ref.at[slot],
        ).wait()
        return carry

    jax.lax.fori_loop(_NCH - _K, _NCH, drain, 0)


def _sc_body(pe01_hbm, pe2_hbm, recip_hbm, out_hbm,
             pe2_v, pe01_v, recip_v, buf0_v, buf1_v, sems):
    wid = lax.axis_index("s") * 2 + lax.axis_index("c")
    base = wid * _RPW

    pltpu.sync_copy(pe2_hbm, pe2_v)
    pltpu.sync_copy(pe01_hbm.at[pl.ds(base, _RPW)], pe01_v)
    pltpu.sync_copy(recip_hbm.at[pl.ds(base * _C, _RPW * _C)], recip_v)

    def compute_row(i, buf):
        pe01_row = [pe01_v[i, pl.ds(j * 16, 16)] for j in range(_D // 16)]

        @plsc.parallel_loop(0, _C // 16)
        def _c16_body(cb):
            c0 = cb * 16
            rvec = recip_v[pl.ds(i * _C + c0, 16)]
            for l in range(16):
                rb = jnp.full((16,), rvec[l])
                for j in range(_D // 16):
                    sl = pl.ds(j * 16, 16)
                    buf[c0 + l, sl] = (pe01_row[j] + pe2_v[c0 + l, sl]) * rb

    def pair_body(p, carry):
        i0 = 2 * p

        @pl.when(p >= 1)
        def _():
            pltpu.make_async_copy(
                buf0_v, out_hbm.at[base + i0 - 2], sems.at[0]
            ).wait()

        compute_row(i0, buf0_v)
        pltpu.make_async_copy(buf0_v, out_hbm.at[base + i0], sems.at[0]).start()

        @pl.when(p >= 1)
        def _():
            pltpu.make_async_copy(
                buf1_v, out_hbm.at[base + i0 - 1], sems.at[1]
            ).wait()

        compute_row(i0 + 1, buf1_v)
        pltpu.make_async_copy(
            buf1_v, out_hbm.at[base + i0 + 1], sems.at[1]
        ).start()
        return carry

    lax.fori_loop(0, _RPW // 2, pair_body, 0)

    pltpu.make_async_copy(buf0_v, out_hbm.at[base + _RPW - 2], sems.at[0]).wait()
    pltpu.make_async_copy(buf1_v, out_hbm.at[base + _RPW - 1], sems.at[1]).wait()


_sc_kernel = functools.partial(
    pl.kernel,
    out_type=jax.ShapeDtypeStruct((_SC_ROWS, _C, _D), jnp.float32),
    mesh=plsc.VectorSubcoreMesh(core_axis_name="c", subcore_axis_name="s"),
    scratch_types=[
        pltpu.MemorySpace.VMEM((_C, _D), jnp.float32),
        pltpu.MemorySpace.VMEM((_RPW, _D), jnp.float32),
        pltpu.MemorySpace.VMEM((_RPW * _C,), jnp.float32),
        pltpu.MemorySpace.VMEM((_C, _D), jnp.float32),
        pltpu.MemorySpace.VMEM((_C, _D), jnp.float32),
        pltpu.SemaphoreType.DMA((2,)),
    ],
)(_sc_body)


def kernel(pos_embed_0, pos_embed_1, pos_embed_2, axial0, axial1, axial2):
    pe0 = pos_embed_0[:_A]
    pe1 = pos_embed_1[:_B]
    pe2 = pos_embed_2[:_C]

    recip, pe01 = pl.pallas_call(
        _recip_kernel,
        out_shape=[
            jax.ShapeDtypeStruct((_ROWS, _C), jnp.float32),
            jax.ShapeDtypeStruct((_ROWS, _D), jnp.float32),
        ],
    )(pe0, pe1, pe2)

    out_sc = _sc_kernel(
        pe01[_TC_ROWS:], pe2, recip[_TC_ROWS:].reshape(_SC_ROWS * _C)
    )

    out_tc = pl.pallas_call(
        _wide_kernel,
        in_specs=[
            pl.BlockSpec(memory_space=pltpu.MemorySpace.VMEM),
            pl.BlockSpec(memory_space=pltpu.MemorySpace.VMEM),
            pl.BlockSpec(memory_space=pltpu.MemorySpace.VMEM),
            pl.BlockSpec(memory_space=pltpu.MemorySpace.VMEM),
        ],
        out_specs=pl.BlockSpec(memory_space=pltpu.MemorySpace.HBM),
        out_shape=jax.ShapeDtypeStruct((_TC_ROWS, _C, _D), jnp.float32),
        scratch_shapes=[
            pltpu.MemorySpace.VMEM((_ROWS, _D), jnp.float32),
            pltpu.MemorySpace.VMEM((_K, _CH, _C, _D), jnp.float32),
            pltpu.SemaphoreType.DMA((_K,)),
        ],
    )(pe0, pe1, pe2, recip)

    out = jnp.concatenate([out_tc, out_sc], axis=0)
    return out.reshape(_A, _B, _C, _D)


# speedup vs baseline: 5.4705x; 3.3973x over previous
import jax
import jax.numpy as jnp
from jax.experimental import pallas as pl
from jax.experimental.pallas import tpu as pltpu

_A, _B, _C, _D = 16, 128, 128, 256
_EPS = 1e-6
_ROWS = _A * _B
_CH = 32
_NCH = _ROWS // _CH
_K = 4


def _wide_kernel(pe0_ref, pe1_ref, pe2_ref, out_ref, pe01_ref, buf_ref, sem_ref):
    pe0 = pe0_ref[0:_A, :]
    pe1 = pe1_ref[0:_B, :]
    pe2 = pe2_ref[0:_C, :]
    pe01_ref[:, :] = (pe0[:, None, :] + pe1[None, :, :]).reshape(_ROWS, _D)
    pe2b = pe2.astype(jnp.bfloat16)
    n2 = jnp.sum(pe2 * pe2, axis=-1)

    def body(i, carry):
        slot = jax.lax.rem(i, _K)

        @pl.when(i >= _K)
        def _():
            pltpu.make_async_copy(
                buf_ref.at[slot],
                out_ref.at[pl.ds((i - _K) * _CH, _CH)],
                sem_ref.at[slot],
            ).wait()

        rows = pe01_ref[pl.ds(i * _CH, _CH), :]
        n01 = jnp.sum(rows * rows, axis=-1, keepdims=True)
        dots = jax.lax.dot_general(
            rows.astype(jnp.bfloat16),
            pe2b,
            (((1,), (1,)), ((), ())),
            preferred_element_type=jnp.float32,
        )
        ssq = n01 + 2.0 * dots + n2[None, :]
        recip = 1.0 / (_EPS + jnp.sqrt(ssq) * (1.0 / 16.0))
        buf_ref[slot] = (rows[:, None, :] + pe2[None, :, :]) * recip[:, :, None]
        pltpu.make_async_copy(
            buf_ref.at[slot],
            out_ref.at[pl.ds(i * _CH, _CH)],
            sem_ref.at[slot],
        ).start()
        return carry

    jax.lax.fori_loop(0, _NCH, body, 0)

    def drain(j, carry):
        pltpu.make_async_copy(
            buf_ref.at[jax.lax.rem(j, _K)],
            out_ref.at[pl.ds(j * _CH, _CH)],
            sem_ref.at[jax.lax.rem(j, _K)],
        ).wait()
        return carry

    jax.lax.fori_loop(_NCH - _K, _NCH, drain, 0)


def kernel(pos_embed_0, pos_embed_1, pos_embed_2, axial0, axial1, axial2):
    out = pl.pallas_call(
        _wide_kernel,
        in_specs=[
            pl.BlockSpec(memory_space=pltpu.MemorySpace.VMEM),
            pl.BlockSpec(memory_space=pltpu.MemorySpace.VMEM),
            pl.BlockSpec(memory_space=pltpu.MemorySpace.VMEM),
        ],
        out_specs=pl.BlockSpec(memory_space=pltpu.MemorySpace.HBM),
        out_shape=jax.ShapeDtypeStruct((_ROWS, _C, _D), jnp.float32),
        scratch_shapes=[
            pltpu.MemorySpace.VMEM((_ROWS, _D), jnp.float32),
            pltpu.MemorySpace.VMEM((_K, _CH, _C, _D), jnp.float32),
            pltpu.SemaphoreType.DMA((_K,)),
        ],
    )(pos_embed_0, pos_embed_1, pos_embed_2)
    return out.reshape(_A, _B, _C, _D)
